# baseline (speedup 1.0000x reference)
import functools

import jax
import jax.numpy as jnp
from jax import lax
from jax.experimental import pallas as pl
from jax.experimental.pallas import tpu as pltpu
from jax.experimental.pallas import tpu_sc as plsc

D = 64
NC, NS = 2, 16
NW = NC * NS
CHUNK = 512
SUB = 128


def _make_gather(batch):
  assert batch % (NW * CHUNK) == 0
  b_per_w = batch // NW
  n_chunks = b_per_w // CHUNK
  mesh = plsc.VectorSubcoreMesh(core_axis_name="c", subcore_axis_name="s")

  @functools.partial(
      pl.kernel,
      mesh=mesh,
      out_type=jax.ShapeDtypeStruct((batch, D), jnp.float32),
      scratch_types=[
          pltpu.VMEM((CHUNK,), jnp.int32),
          pltpu.VMEM((CHUNK, D), jnp.float32),
          pltpu.SemaphoreType.DMA,
      ],
      compiler_params=pltpu.CompilerParams(use_tc_tiling_on_sc=False),
  )
  def gather_kernel(table_hbm, idx_hbm, out_hbm, idx_v, rows_v, sem):
    wid = lax.axis_index("s") * NC + lax.axis_index("c")
    base = wid * b_per_w

    def body(g, carry):
      off = base + g * CHUNK
      pltpu.sync_copy(idx_hbm.at[pl.ds(off, CHUNK)], idx_v)
      copies = []
      for j in range(CHUNK // SUB):
        copies.append(
            pltpu.async_copy(
                table_hbm.at[idx_v.at[pl.ds(j * SUB, SUB)]],
                rows_v.at[pl.ds(j * SUB, SUB)],
                sem,
            ))
      for c in copies:
        c.wait()
      pltpu.sync_copy(rows_v, out_hbm.at[pl.ds(off, CHUNK)])
      return carry

    lax.fori_loop(0, n_chunks, body, 0)

  return gather_kernel


def kernel(indices, table):
  batch = indices.shape[0] * indices.shape[1]
  idx_flat = indices.reshape(batch).astype(jnp.int32)
  out = _make_gather(batch)(table, idx_flat)
  return out.reshape(indices.shape + (D,))


# speedup vs baseline: 1.0430x; 1.0430x over previous
import functools

import jax
import jax.numpy as jnp
from jax import lax
from jax.experimental import pallas as pl
from jax.experimental.pallas import tpu as pltpu
from jax.experimental.pallas import tpu_sc as plsc

D = 64
NC, NS = 2, 16
NW = NC * NS
CHUNK = 512
SUB = 128
NSUB = CHUNK // SUB


def _make_gather(batch):
  assert batch % (NW * CHUNK) == 0
  b_per_w = batch // NW
  n_chunks = b_per_w // CHUNK
  assert n_chunks >= 4
  mesh = plsc.VectorSubcoreMesh(core_axis_name="c", subcore_axis_name="s")

  @functools.partial(
      pl.kernel,
      mesh=mesh,
      out_type=jax.ShapeDtypeStruct((batch, D), jnp.float32),
      scratch_types=[
          pltpu.VMEM((2, CHUNK), jnp.int32),
          pltpu.VMEM((2, CHUNK, D), jnp.float32),
          pltpu.SemaphoreType.DMA((2,)),
          pltpu.SemaphoreType.DMA((2,)),
          pltpu.SemaphoreType.DMA((2,)),
      ],
      compiler_params=pltpu.CompilerParams(use_tc_tiling_on_sc=False),
  )
  def gather_kernel(table_hbm, idx_hbm, out_hbm, idx_v, rows_v, sem_i,
                    sem_g, sem_w):
    wid = lax.axis_index("s") * NC + lax.axis_index("c")
    base = wid * b_per_w

    def fire_idx(g, b):
      off = base + g * CHUNK
      pltpu.async_copy(idx_hbm.at[pl.ds(off, CHUNK)], idx_v.at[b],
                       sem_i.at[b])

    def wait_idx(b):
      pltpu.make_async_copy(idx_hbm.at[pl.ds(0, CHUNK)], idx_v.at[b],
                            sem_i.at[b]).wait()

    def fire_gathers(b):
      for j in range(NSUB):
        pltpu.async_copy(
            table_hbm.at[idx_v.at[b].at[pl.ds(j * SUB, SUB)]],
            rows_v.at[b].at[pl.ds(j * SUB, SUB)],
            sem_g.at[b])

    def wait_gathers(b):
      for j in range(NSUB):
        pltpu.make_async_copy(
            table_hbm.at[idx_v.at[b].at[pl.ds(j * SUB, SUB)]],
            rows_v.at[b].at[pl.ds(j * SUB, SUB)],
            sem_g.at[b]).wait()

    def fire_wb(g, b):
      off = base + g * CHUNK
      pltpu.async_copy(rows_v.at[b], out_hbm.at[pl.ds(off, CHUNK)],
                       sem_w.at[b])

    def wait_wb(b):
      pltpu.make_async_copy(rows_v.at[b], out_hbm.at[pl.ds(0, CHUNK)],
                            sem_w.at[b]).wait()

    fire_idx(0, 0)
    fire_idx(1, 1)
    wait_idx(0)
    fire_gathers(0)
    wait_idx(1)
    fire_gathers(1)
    wait_gathers(0)
    fire_wb(0, 0)
    fire_idx(2, 0)
    wait_idx(0)
    wait_wb(0)
    fire_gathers(0)
    wait_gathers(1)
    fire_wb(1, 1)
    fire_idx(3, 1)

    def body(g, carry):
      b = lax.rem(g, 2)
      wait_idx(b)
      wait_wb(b)
      fire_gathers(b)
      nb = 1 - b
      wait_gathers(nb)
      fire_wb(g - 1, nb)
      fire_idx(g + 1, nb)
      return carry

    lax.fori_loop(3, n_chunks - 1, body, 0)

    gl = n_chunks - 1
    bl = gl % 2
    wait_idx(bl)
    wait_wb(bl)
    fire_gathers(bl)
    wait_gathers(1 - bl)
    fire_wb(gl - 1, 1 - bl)
    wait_gathers(bl)
    fire_wb(gl, bl)
    wait_wb(1 - bl)
    wait_wb(bl)

  return gather_kernel


def kernel(indices, table):
  batch = indices.shape[0] * indices.shape[1]
  idx_flat = indices.reshape(batch).astype(jnp.int32)
  out = _make_gather(batch)(table, idx_flat)
  return out.reshape(indices.shape + (D,))


# speedup vs baseline: 1.0858x; 1.0410x over previous
import functools

import jax
import jax.numpy as jnp
from jax import lax
from jax.experimental import pallas as pl
from jax.experimental.pallas import tpu as pltpu
from jax.experimental.pallas import tpu_sc as plsc

D = 64
NC, NS = 2, 16
NW = NC * NS
CHUNK = 512
SUB = 128
NSUB = CHUNK // SUB

TBLK = 2048


def _table_transpose_body(src_ref, dst_ref):
  x = src_ref[...]
  y = jnp.transpose(x, (1, 0))
  z = y.reshape(TBLK // 2, 2, D)
  dst_ref[...] = jnp.concatenate([z[:, 0, :], z[:, 1, :]], axis=1)


def _transpose_table(t1):
  v = t1.shape[1]
  return pl.pallas_call(
      _table_transpose_body,
      grid=((v + TBLK - 1) // TBLK,),
      in_specs=[pl.BlockSpec((64, TBLK), lambda i: (0, i))],
      out_specs=pl.BlockSpec((TBLK // 2, 128), lambda i: (i, 0)),
      out_shape=jax.ShapeDtypeStruct((v // 2, 128), jnp.float32),
  )(t1)


def _make_gather(batch):
  assert batch % (NW * CHUNK) == 0
  b_per_w = batch // NW
  n_chunks = b_per_w // CHUNK
  assert n_chunks >= 4
  mesh = plsc.VectorSubcoreMesh(core_axis_name="c", subcore_axis_name="s")

  @functools.partial(
      pl.kernel,
      mesh=mesh,
      out_type=jax.ShapeDtypeStruct((batch, D), jnp.float32),
      scratch_types=[
          pltpu.VMEM((2, CHUNK), jnp.int32),
          pltpu.VMEM((2, CHUNK, D), jnp.float32),
          pltpu.SemaphoreType.DMA((2,)),
          pltpu.SemaphoreType.DMA((2,)),
          pltpu.SemaphoreType.DMA((2,)),
      ],
      compiler_params=pltpu.CompilerParams(use_tc_tiling_on_sc=False),
  )
  def gather_kernel(table_hbm, idx_hbm, out_hbm, idx_v, rows_v, sem_i,
                    sem_g, sem_w):
    wid = lax.axis_index("s") * NC + lax.axis_index("c")
    base = wid * b_per_w

    def fire_idx(g, b):
      off = base + g * CHUNK
      pltpu.async_copy(idx_hbm.at[pl.ds(off, CHUNK)], idx_v.at[b],
                       sem_i.at[b])

    def wait_idx(b):
      pltpu.make_async_copy(idx_hbm.at[pl.ds(0, CHUNK)], idx_v.at[b],
                            sem_i.at[b]).wait()

    def fire_gathers(b):
      for j in range(NSUB):
        pltpu.async_copy(
            table_hbm.at[idx_v.at[b].at[pl.ds(j * SUB, SUB)]],
            rows_v.at[b].at[pl.ds(j * SUB, SUB)],
            sem_g.at[b])

    def wait_gathers(b):
      for j in range(NSUB):
        pltpu.make_async_copy(
            table_hbm.at[idx_v.at[b].at[pl.ds(j * SUB, SUB)]],
            rows_v.at[b].at[pl.ds(j * SUB, SUB)],
            sem_g.at[b]).wait()

    def fire_wb(g, b):
      off = base + g * CHUNK
      pltpu.async_copy(rows_v.at[b], out_hbm.at[pl.ds(off, CHUNK)],
                       sem_w.at[b])

    def wait_wb(b):
      pltpu.make_async_copy(rows_v.at[b], out_hbm.at[pl.ds(0, CHUNK)],
                            sem_w.at[b]).wait()

    fire_idx(0, 0)
    fire_idx(1, 1)
    wait_idx(0)
    fire_gathers(0)
    wait_idx(1)
    fire_gathers(1)
    wait_gathers(0)
    fire_wb(0, 0)
    fire_idx(2, 0)
    wait_idx(0)
    wait_wb(0)
    fire_gathers(0)
    wait_gathers(1)
    fire_wb(1, 1)
    fire_idx(3, 1)

    def body(g, carry):
      b = lax.rem(g, 2)
      wait_idx(b)
      wait_wb(b)
      fire_gathers(b)
      nb = 1 - b
      wait_gathers(nb)
      fire_wb(g - 1, nb)
      fire_idx(g + 1, nb)
      return carry

    lax.fori_loop(3, n_chunks - 1, body, 0)

    gl = n_chunks - 1
    bl = gl % 2
    wait_idx(bl)
    wait_wb(bl)
    fire_gathers(bl)
    wait_gathers(1 - bl)
    fire_wb(gl - 1, 1 - bl)
    wait_gathers(bl)
    fire_wb(gl, bl)
    wait_wb(1 - bl)
    wait_wb(bl)

  return gather_kernel


def _detranspose_body(src_ref, dst_ref):
  x = src_ref[...]
  ya = jnp.transpose(x[:, :D], (1, 0))
  yb = jnp.transpose(x[:, D:], (1, 0))
  dst_ref[0] = jnp.concatenate([ya, yb], axis=1)


def _detranspose(rows2, t, b):
  return pl.pallas_call(
      _detranspose_body,
      grid=(t,),
      in_specs=[pl.BlockSpec((b // 2, 128), lambda i: (i, 0))],
      out_specs=pl.BlockSpec((1, D, b), lambda i: (i, 0, 0)),
      out_shape=jax.ShapeDtypeStruct((t, D, b), jnp.float32),
  )(rows2)


def kernel(indices, table):
  b, t = indices.shape
  v = table.shape[0]
  batch = b * t
  table_rm = _transpose_table(jnp.transpose(table))
  table_lin = table_rm.reshape(v, D)
  idx_perm = jnp.transpose(
      indices.reshape(2, b // 2, t), (2, 1, 0)).reshape(batch)
  idx_flat = idx_perm.astype(jnp.int32)
  rows = _make_gather(batch)(table_lin, idx_flat)
  out3 = _detranspose(rows.reshape(batch // 2, 128), t, b)
  return jnp.transpose(out3, (2, 0, 1))
